# baseline (speedup 1.0000x reference)
import functools

import jax
import jax.numpy as jnp
from jax.experimental import pallas as pl
from jax.experimental.pallas import tpu as pltpu

_ROWS = 200


def _cheb_kernel(gso_top_ref, gso_bot_ref, x_full_ref, w0_ref, w1_ref,
                 bias_ref, out_ref, xw1_ref, small_ref):
    i = pl.program_id(0)
    half_tiles = pl.num_programs(0) // 2

    @pl.when(i == 0)
    def _init():
        xw1_ref[...] = jnp.dot(x_full_ref[...], w1_ref[...],
                               preferred_element_type=jnp.float32
                               ).astype(jnp.bfloat16)
        small_ref[...] = (jnp.dot(x_full_ref[...], w0_ref[...],
                                  preferred_element_type=jnp.float32)
                          + bias_ref[...])

    row_tile = (i % 2) * half_tiles + i // 2
    small = small_ref[pl.ds(row_tile * _ROWS, _ROWS), :]

    @pl.when(i % 2 == 0)
    def _top():
        out_ref[...] = small + jnp.dot(
            gso_top_ref[...].astype(jnp.bfloat16), xw1_ref[...],
            preferred_element_type=jnp.float32)

    @pl.when(i % 2 == 1)
    def _bot():
        out_ref[...] = small + jnp.dot(
            gso_bot_ref[...].astype(jnp.bfloat16), xw1_ref[...],
            preferred_element_type=jnp.float32)


@functools.partial(jax.jit, static_argnames=())
def kernel(x, gso, weight, bias):
    b, n, d_in = x.shape
    d_out = weight.shape[-1]
    x2 = x[0]
    gso2 = gso[0]
    w0 = weight[0]
    w1 = weight[1]
    bias2 = bias.reshape(1, d_out)

    half_tiles = n // (2 * _ROWS)

    def row_tile(i):
        return (i % 2) * half_tiles + i // 2

    out = pl.pallas_call(
        _cheb_kernel,
        grid=(2 * half_tiles,),
        in_specs=[
            pl.BlockSpec((_ROWS, n), lambda i: (i // 2, 0)),
            pl.BlockSpec(
                (_ROWS, n),
                lambda i: (half_tiles + jnp.maximum(i - 1, 0) // 2, 0)),
            pl.BlockSpec((n, d_in), lambda i: (0, 0)),
            pl.BlockSpec((d_in, d_out), lambda i: (0, 0)),
            pl.BlockSpec((d_in, d_out), lambda i: (0, 0)),
            pl.BlockSpec((1, d_out), lambda i: (0, 0)),
        ],
        out_specs=pl.BlockSpec((_ROWS, d_out), lambda i: (row_tile(i), 0)),
        out_shape=jax.ShapeDtypeStruct((n, d_out), jnp.float32),
        scratch_shapes=[pltpu.VMEM((n, d_out), jnp.bfloat16),
                        pltpu.VMEM((n, d_out), jnp.float32)],
    )(gso2, gso2, x2, w0, w1, bias2)
    return out.reshape(b, n, d_out)


# speedup vs baseline: 1.0400x; 1.0400x over previous
import functools

import jax
import jax.numpy as jnp
from jax.experimental import pallas as pl
from jax.experimental.pallas import tpu as pltpu

_ROWS = 400


def _cheb_kernel(gso_ref, x_full_ref, w0_ref, w1_ref, bias_ref,
                 out_ref, xw1_ref, small_ref):
    i = pl.program_id(0)

    @pl.when(i == 0)
    def _init():
        xw1_ref[...] = jnp.dot(x_full_ref[...], w1_ref[...],
                               preferred_element_type=jnp.float32
                               ).astype(jnp.bfloat16)
        small_ref[...] = (jnp.dot(x_full_ref[...], w0_ref[...],
                                  preferred_element_type=jnp.float32)
                          + bias_ref[...])

    out_ref[...] = (
        small_ref[pl.ds(i * _ROWS, _ROWS), :]
        + jnp.dot(gso_ref[...].astype(jnp.bfloat16), xw1_ref[...],
                  preferred_element_type=jnp.float32)
    )


@functools.partial(jax.jit, static_argnames=())
def kernel(x, gso, weight, bias):
    b, n, d_in = x.shape
    d_out = weight.shape[-1]
    x2 = x[0]
    gso2 = gso[0]
    w0 = weight[0]
    w1 = weight[1]
    bias2 = bias.reshape(1, d_out)

    grid = (n // _ROWS,)
    out = pl.pallas_call(
        _cheb_kernel,
        grid=grid,
        in_specs=[
            pl.BlockSpec((_ROWS, n), lambda i: (i, 0)),
            pl.BlockSpec((n, d_in), lambda i: (0, 0)),
            pl.BlockSpec((d_in, d_out), lambda i: (0, 0)),
            pl.BlockSpec((d_in, d_out), lambda i: (0, 0)),
            pl.BlockSpec((1, d_out), lambda i: (0, 0)),
        ],
        out_specs=pl.BlockSpec((_ROWS, d_out), lambda i: (i, 0)),
        out_shape=jax.ShapeDtypeStruct((n, d_out), jnp.float32),
        scratch_shapes=[pltpu.VMEM((n, d_out), jnp.bfloat16),
                        pltpu.VMEM((n, d_out), jnp.float32)],
    )(gso2, x2, w0, w1, bias2)
    return out.reshape(b, n, d_out)
